# Initial kernel scaffold; baseline (speedup 1.0000x reference)
import jax
import jax.numpy as jnp
from jax.experimental import pallas as pl


def kernel(x, edge_index, edge_attr, W_ne, b_ne, W_ee, b_ee, W1a, b1a, W1b, b1b, W2a, b2a, W2b, b2b, W_rp, b_rp):
    raise NotImplementedError("write your pallas kernel here")


# speedup vs baseline: 20.7502x; 20.7502x over previous
import functools

import jax
import jax.numpy as jnp
from jax import lax
from jax.experimental import pallas as pl
from jax.experimental.pallas import tpu as pltpu
from jax.experimental.pallas import tpu_sc as plsc

N = 5
R = 4
H = 64
E = 800000
NTILES = 32
EPW = E // NTILES
ACC = N * H
OUTW = ACC + N * 16


def _make_edge_pass(with_counts: bool):
    mesh = plsc.VectorSubcoreMesh(core_axis_name="c", subcore_axis_name="s")

    @functools.partial(
        pl.kernel,
        out_type=jax.ShapeDtypeStruct((NTILES, OUTW), jnp.float32),
        mesh=mesh,
        scratch_types=[
            pltpu.VMEM((EPW,), jnp.int32),
            pltpu.VMEM((EPW,), jnp.int32),
            pltpu.VMEM((EPW,), jnp.float32),
            pltpu.VMEM((ACC,), jnp.float32),
            pltpu.VMEM((H,), jnp.float32),
            pltpu.VMEM((OUTW,), jnp.float32),
        ],
    )
    def edge_pass(row_hbm, col_hbm, t_hbm, ap_hbm, u_hbm, out_hbm,
                  row_v, col_v, t_v, ap_v, u_v, acc_v):
        wid = lax.axis_index("c") * 16 + lax.axis_index("s")
        base = wid * EPW
        pltpu.sync_copy(row_hbm.at[pl.ds(base, EPW)], row_v)
        pltpu.sync_copy(col_hbm.at[pl.ds(base, EPW)], col_v)
        pltpu.sync_copy(t_hbm.at[pl.ds(base, EPW)], t_v)
        pltpu.sync_copy(ap_hbm, ap_v)
        pltpu.sync_copy(u_hbm, u_v)

        zero = jnp.zeros((16,), jnp.float32)
        for i in range(OUTW // 16):
            acc_v[pl.ds(i * 16, 16)] = zero

        u_regs = [u_v[pl.ds(s * 16, 16)] for s in range(H // 16)]
        ones = jnp.ones((16,), jnp.float32)

        def do_edge(rv, cv, tvv, lane):
            r = rv[lane]
            c = cv[lane]
            tv = tvv[lane]
            rb = r * H
            cb = c * H
            for s in range(H // 16):
                a = ap_v[pl.ds(rb + s * 16, 16)]
                val = jnp.maximum(a + tv * u_regs[s], 0.0)
                plsc.addupdate(acc_v.at[pl.ds(cb + s * 16, 16)], val)
            if with_counts:
                plsc.addupdate(acc_v.at[pl.ds(ACC + c * 16, 16)], ones)

        n_full = EPW // 16

        @plsc.parallel_loop(0, n_full, step=1)
        def _loop(g):
            gb = g * 16
            rv = row_v[pl.ds(gb, 16)]
            cv = col_v[pl.ds(gb, 16)]
            tvv = t_v[pl.ds(gb, 16)]
            for lane in range(16):
                do_edge(rv, cv, tvv, lane)

        n_tail = EPW - n_full * 16
        if n_tail:
            tb = EPW - 16
            rv = row_v[pl.ds(tb, 16)]
            cv = col_v[pl.ds(tb, 16)]
            tvv = t_v[pl.ds(tb, 16)]
            for lane in range(16 - n_tail, 16):
                do_edge(rv, cv, tvv, lane)

        pltpu.sync_copy(acc_v, out_hbm.at[wid])

    return edge_pass


_edge_pass_counts = _make_edge_pass(True)
_edge_pass_plain = _make_edge_pass(False)


def _fold(h, Wa, ba, W_ee, b_ee):
    A = h @ Wa[:H]
    u = W_ee[0] @ Wa[H:]
    cv = b_ee @ Wa[H:] + ba
    return (A + cv[None, :]).reshape(-1), u


def kernel(x, edge_index, edge_attr, W_ne, b_ne, W_ee, b_ee,
           W1a, b1a, W1b, b1b, W2a, b2a, W2b, b2b, W_rp, b_rp):
    row = edge_index[0]
    col = edge_index[1]
    t = edge_attr[:, 0]

    x1 = x @ W_ne + b_ne

    ap1, u1 = _fold(x1, W1a, b1a, W_ee, b_ee)
    out1 = _edge_pass_counts(row, col, t, ap1, u1)
    S1 = out1[:, :ACC].sum(axis=0).reshape(N, H)
    cnt = out1[:, ACC:OUTW:16].sum(axis=0)
    m1 = S1 @ W1b + cnt[:, None] * b1b
    x2 = x1 + m1

    ap2, u2 = _fold(x2, W2a, b2a, W_ee, b_ee)
    out2 = _edge_pass_plain(row, col, t, ap2, u2)
    S2 = out2[:, :ACC].sum(axis=0).reshape(N, H)
    m2 = S2 @ W2b + cnt[:, None] * b2b
    x3 = x2 + m2

    role_logits = x3 @ W_rp + b_rp
    return jax.nn.softmax(role_logits, axis=-1)


# speedup vs baseline: 23.0646x; 1.1115x over previous
import functools

import jax
import jax.numpy as jnp
from jax import lax
from jax.experimental import pallas as pl
from jax.experimental.pallas import tpu as pltpu
from jax.experimental.pallas import tpu_sc as plsc

N = 5
R = 4
H = 64
E = 800000
NTILES = 32
EPW = E // NTILES
ACC = N * H
OUTW = ACC + N * 16


def _make_edge_pass(with_counts: bool):
    mesh = plsc.VectorSubcoreMesh(core_axis_name="c", subcore_axis_name="s")

    @functools.partial(
        pl.kernel,
        out_type=jax.ShapeDtypeStruct((NTILES, OUTW), jnp.float32),
        mesh=mesh,
        scratch_types=[
            pltpu.VMEM((EPW,), jnp.int32),
            pltpu.VMEM((EPW,), jnp.int32),
            pltpu.VMEM((EPW,), jnp.float32),
            pltpu.VMEM((ACC,), jnp.float32),
            pltpu.VMEM((H,), jnp.float32),
            pltpu.VMEM((OUTW,), jnp.float32),
        ],
    )
    def edge_pass(ei_hbm, t_hbm, ap_hbm, u_hbm, out_hbm,
                  row_v, col_v, t_v, ap_v, u_v, acc_v):
        wid = lax.axis_index("c") * 16 + lax.axis_index("s")
        base = wid * EPW
        pltpu.sync_copy(ei_hbm.at[pl.ds(base, EPW)], row_v)
        pltpu.sync_copy(ei_hbm.at[pl.ds(E + base, EPW)], col_v)
        pltpu.sync_copy(t_hbm.at[pl.ds(base, EPW)], t_v)
        pltpu.sync_copy(ap_hbm, ap_v)
        pltpu.sync_copy(u_hbm, u_v)

        zero = jnp.zeros((16,), jnp.float32)
        for i in range(OUTW // 16):
            acc_v[pl.ds(i * 16, 16)] = zero

        u_regs = [u_v[pl.ds(s * 16, 16)] for s in range(H // 16)]

        def do_edge(rv, cv, tvv, lane):
            r = rv[lane]
            c = cv[lane]
            tv = tvv[lane]
            rb = r * H
            cb = c * H
            for s in range(H // 16):
                a = ap_v[pl.ds(rb + s * 16, 16)]
                val = jnp.maximum(a + tv * u_regs[s], 0.0)
                plsc.addupdate(acc_v.at[pl.ds(cb + s * 16, 16)], val)

        n_full = EPW // 16

        onesv = jnp.ones((16,), jnp.float32)
        zerosv = jnp.zeros((16,), jnp.float32)

        def count_cols(cv, valid=None):
            for c in range(N):
                m = cv == c
                if valid is not None:
                    m = m & valid
                plsc.addupdate(acc_v.at[pl.ds(ACC + c * 16, 16)],
                               jnp.where(m, onesv, zerosv))

        @plsc.parallel_loop(0, n_full, step=1)
        def _loop(g):
            gb = g * 16
            rv = row_v[pl.ds(gb, 16)]
            cv = col_v[pl.ds(gb, 16)]
            tvv = t_v[pl.ds(gb, 16)]
            for lane in range(16):
                do_edge(rv, cv, tvv, lane)
            if with_counts:
                count_cols(cv)

        n_tail = EPW - n_full * 16
        if n_tail:
            tb = EPW - 16
            rv = row_v[pl.ds(tb, 16)]
            cv = col_v[pl.ds(tb, 16)]
            tvv = t_v[pl.ds(tb, 16)]
            for lane in range(16 - n_tail, 16):
                do_edge(rv, cv, tvv, lane)
            if with_counts:
                lane_ok = lax.iota(jnp.int32, 16) >= (16 - n_tail)
                count_cols(cv, valid=lane_ok)

        pltpu.sync_copy(acc_v, out_hbm.at[wid])

    return edge_pass


_edge_pass_counts = _make_edge_pass(True)
_edge_pass_plain = _make_edge_pass(False)


def _fold(h, Wa, ba, W_ee, b_ee):
    A = h @ Wa[:H]
    u = W_ee[0] @ Wa[H:]
    cv = b_ee @ Wa[H:] + ba
    return (A + cv[None, :]).reshape(-1), u


def kernel(x, edge_index, edge_attr, W_ne, b_ne, W_ee, b_ee,
           W1a, b1a, W1b, b1b, W2a, b2a, W2b, b2b, W_rp, b_rp):
    ei_flat = edge_index.reshape(2 * E)
    t = edge_attr.reshape(E)

    x1 = x @ W_ne + b_ne

    ap1, u1 = _fold(x1, W1a, b1a, W_ee, b_ee)
    out1 = _edge_pass_counts(ei_flat, t, ap1, u1)
    S1 = out1[:, :ACC].sum(axis=0).reshape(N, H)
    cnt = out1[:, ACC:].reshape(-1, N, 16).sum(axis=(0, 2))
    m1 = S1 @ W1b + cnt[:, None] * b1b
    x2 = x1 + m1

    ap2, u2 = _fold(x2, W2a, b2a, W_ee, b_ee)
    out2 = _edge_pass_plain(ei_flat, t, ap2, u2)
    S2 = out2[:, :ACC].sum(axis=0).reshape(N, H)
    m2 = S2 @ W2b + cnt[:, None] * b2b
    x3 = x2 + m2

    role_logits = x3 @ W_rp + b_rp
    return jax.nn.softmax(role_logits, axis=-1)
